# Initial kernel scaffold; baseline (speedup 1.0000x reference)
import jax
import jax.numpy as jnp
from jax.experimental import pallas as pl


def kernel(features, edge_source, edge_target, weight):
    raise NotImplementedError("write your pallas kernel here")


# speedup vs baseline: 5.5018x; 5.5018x over previous
import functools

import jax
import jax.numpy as jnp
from jax import lax
from jax.experimental import pallas as pl
from jax.experimental.pallas import tpu as pltpu
from jax.experimental.pallas import tpu_sc as plsc

N_NODES = 10000
D_IN = 128
D_OUT = 128

N_ACC = 10240
NC = 2
NS = 16
NW = NC * NS
CHUNK = 128
LANES = 16
TC_BLK = 2048


def _sc_segment_sums(features, src_r, tgt_r, n_chunks):
    mesh = plsc.VectorSubcoreMesh(
        core_axis_name="c", subcore_axis_name="s", num_cores=NC, num_subcores=NS)

    rows_per_tile = N_ACC // NS

    @functools.partial(
        pl.kernel,
        out_type=(jax.ShapeDtypeStruct((NC, N_ACC, D_IN), jnp.float32),
                  jax.ShapeDtypeStruct((NW, 1, N_ACC), jnp.float32)),
        mesh=mesh,
        compiler_params=pltpu.CompilerParams(needs_layout_passes=False),
        scratch_types=[
            pltpu.VMEM((n_chunks, CHUNK), jnp.int32),
            pltpu.VMEM((n_chunks, CHUNK), jnp.int32),
            pltpu.VMEM((CHUNK, D_IN), jnp.float32),
            pltpu.VMEM((N_ACC,), jnp.float32),
            pltpu.VMEM_SHARED((N_ACC, D_IN), jnp.float32),
            pltpu.SemaphoreType.DMA,
        ],
    )
    def seg_kernel(feat_hbm, src_hbm, tgt_hbm, sums_hbm, cnt_hbm,
                   src_v, tgt_v, rows_v, cnt_v, acc_sh, sem):
        c = lax.axis_index("c")
        s = lax.axis_index("s")
        wid = c * NS + s

        pltpu.sync_copy(src_hbm.at[wid], src_v)
        pltpu.sync_copy(tgt_hbm.at[wid], tgt_v)

        zeros16 = jnp.zeros((LANES,), jnp.float32)
        ones16 = jnp.ones((LANES,), jnp.float32)

        def zero_cnt(i, carry):
            cnt_v[pl.ds(i * LANES, LANES)] = zeros16
            return carry

        lax.fori_loop(0, N_ACC // LANES, zero_cnt, 0)

        def zero_row(i, carry):
            for j in range(D_IN // LANES):
                rows_v[i, pl.ds(j * LANES, LANES)] = zeros16
            return carry

        lax.fori_loop(0, CHUNK, zero_row, 0)
        for k in range(rows_per_tile // CHUNK):
            pltpu.sync_copy(
                rows_v, acc_sh.at[pl.ds(s * rows_per_tile + k * CHUNK, CHUNK)])
        plsc.subcore_barrier()

        def body(j, carry):
            pltpu.async_copy(feat_hbm.at[tgt_v.at[j]], rows_v, sem).wait()
            pltpu.sync_copy(rows_v, acc_sh.at[src_v.at[j]], add=True)
            for k in range(CHUNK // LANES):
                idx = src_v[j, pl.ds(k * LANES, LANES)]
                plsc.addupdate_scatter(cnt_v, [idx], ones16)
            return carry

        lax.fori_loop(0, n_chunks, body, 0)
        plsc.subcore_barrier()

        pltpu.sync_copy(cnt_v, cnt_hbm.at[wid, 0])
        pltpu.sync_copy(
            acc_sh.at[pl.ds(s * rows_per_tile, rows_per_tile)],
            sums_hbm.at[c, pl.ds(s * rows_per_tile, rows_per_tile)])

    return seg_kernel(features, src_r, tgt_r)


def _tc_combine(features, weight, sums, counts):
    blk = TC_BLK
    grid = -(-N_NODES // blk)

    def tc_body(feat_ref, w_ref, p_ref, c_ref, out_ref):
        w = w_ref[...]
        nodes = jnp.dot(feat_ref[...], w, preferred_element_type=jnp.float32)
        p = p_ref[0] + p_ref[1]
        cnt = jnp.sum(c_ref[:, 0, :], axis=0)
        mean = p / jnp.maximum(cnt, 1.0)[:, None]
        agg = jnp.dot(mean, w, preferred_element_type=jnp.float32)
        out_ref[...] = jnp.concatenate([nodes, agg], axis=1)

    return pl.pallas_call(
        tc_body,
        grid=(grid,),
        in_specs=[
            pl.BlockSpec((blk, D_IN), lambda i: (i, 0)),
            pl.BlockSpec((D_IN, D_OUT), lambda i: (0, 0)),
            pl.BlockSpec((NC, blk, D_IN), lambda i: (0, i, 0)),
            pl.BlockSpec((NW, 1, blk), lambda i: (0, 0, i)),
        ],
        out_specs=pl.BlockSpec((blk, 2 * D_OUT), lambda i: (i, 0)),
        out_shape=jax.ShapeDtypeStruct((N_NODES, 2 * D_OUT), jnp.float32),
    )(features, weight, sums, counts)


def kernel(features, edge_source, edge_target, weight):
    n_edges = edge_source.shape[0]
    per_w = -(-n_edges // (NW * CHUNK)) * CHUNK
    e_pad = per_w * NW
    n_chunks = per_w // CHUNK

    pad = e_pad - n_edges
    src_r = jnp.concatenate(
        [edge_source, jnp.full((pad,), N_NODES, jnp.int32)]).reshape(
            NW, n_chunks, CHUNK)
    tgt_r = jnp.concatenate(
        [edge_target, jnp.zeros((pad,), jnp.int32)]).reshape(
            NW, n_chunks, CHUNK)

    sums, counts = _sc_segment_sums(features, src_r, tgt_r, n_chunks)
    return _tc_combine(features, weight, sums, counts)
